# baseline (device time: 38099 ns/iter reference)
import jax
import jax.numpy as jnp
from jax import lax
from jax.experimental import pallas as pl
from jax.experimental.pallas import tpu as pltpu

N_DEV = 4


def kernel(x, Wq, Wo, K_ext, V_ext):
    B, Sq, D = x.shape
    _, skv, Hq, Dh = K_ext.shape

    kT = jnp.transpose(K_ext, (0, 2, 3, 1))
    vT = jnp.transpose(V_ext, (0, 2, 3, 1))

    def body(x_ref, wq_ref, wo_ref, k_ref, v_ref, out_ref,
             kfull, vfull, ksend, krecv, vsend, vrecv):
        my = lax.axis_index("i")
        left = lax.rem(my + (N_DEV - 1), N_DEV)
        right = lax.rem(my + 1, N_DEV)

        barrier = pltpu.get_barrier_semaphore()
        for nbr in (left, right):
            pl.semaphore_signal(barrier, inc=1, device_id=(nbr,),
                                device_id_type=pl.DeviceIdType.MESH)
        pl.semaphore_wait(barrier, 2)

        kfull[0] = k_ref[...].astype(jnp.bfloat16)
        vfull[0] = v_ref[...].astype(jnp.bfloat16)

        def hop(h):
            kr = pltpu.make_async_remote_copy(
                src_ref=kfull.at[h], dst_ref=kfull.at[h + 1],
                send_sem=ksend.at[h], recv_sem=krecv.at[h],
                device_id=(right,), device_id_type=pl.DeviceIdType.MESH)
            vr = pltpu.make_async_remote_copy(
                src_ref=vfull.at[h], dst_ref=vfull.at[h + 1],
                send_sem=vsend.at[h], recv_sem=vrecv.at[h],
                device_id=(right,), device_id_type=pl.DeviceIdType.MESH)
            return kr, vr

        kr0, vr0 = hop(0)
        kr0.start()
        vr0.start()

        xb = x_ref[...].reshape(B * Sq, D).astype(jnp.bfloat16)
        q2 = jnp.dot(xb, wq_ref[...].astype(jnp.bfloat16),
                     preferred_element_type=jnp.float32)

        kr0.wait()
        vr0.wait()
        for h in range(1, N_DEV - 1):
            kr, vr = hop(h)
            kr.start()
            vr.start()
            kr.wait()
            vr.wait()

        wo = wo_ref[...].astype(jnp.bfloat16)
        for b in range(B):
            acc = jnp.zeros((Sq, D), jnp.float32)
            for hh in range(Hq):
                q_bh = q2[b * Sq:(b + 1) * Sq, hh * Dh:(hh + 1) * Dh]
                kT_bh = jnp.concatenate(
                    [kfull[s, b, hh, :, :] for s in range(N_DEV)], axis=-1)
                vT_bh = jnp.concatenate(
                    [vfull[s, b, hh, :, :] for s in range(N_DEV)], axis=-1)
                s = lax.dot_general(
                    q_bh.astype(jnp.bfloat16), kT_bh,
                    (((1,), (0,)), ((), ())),
                    preferred_element_type=jnp.float32) * 0.125
                m = jnp.max(s, axis=-1, keepdims=True)
                p = jnp.exp(s - m)
                l = jnp.sum(p, axis=-1, keepdims=True)
                o = lax.dot_general(
                    p.astype(jnp.bfloat16), vT_bh,
                    (((1,), (1,)), ((), ())),
                    preferred_element_type=jnp.float32) / l
                acc = acc + lax.dot_general(
                    o.astype(jnp.bfloat16), wo[hh * Dh:(hh + 1) * Dh, :],
                    (((1,), (0,)), ((), ())),
                    preferred_element_type=jnp.float32)
            out_ref[b] = acc

    return pl.pallas_call(
        body,
        out_shape=jax.ShapeDtypeStruct((B, Sq, D), jnp.float32),
        in_specs=[pl.BlockSpec(memory_space=pltpu.VMEM)] * 5,
        out_specs=pl.BlockSpec(memory_space=pltpu.VMEM),
        scratch_shapes=[
            pltpu.VMEM((N_DEV, B, Hq, Dh, skv), jnp.bfloat16),
            pltpu.VMEM((N_DEV, B, Hq, Dh, skv), jnp.bfloat16),
            pltpu.SemaphoreType.DMA((N_DEV - 1,)),
            pltpu.SemaphoreType.DMA((N_DEV - 1,)),
            pltpu.SemaphoreType.DMA((N_DEV - 1,)),
            pltpu.SemaphoreType.DMA((N_DEV - 1,)),
        ],
        compiler_params=pltpu.CompilerParams(collective_id=0),
    )(x, Wq, Wo, kT, vT)


# device time: 20251 ns/iter; 1.8813x vs baseline; 1.8813x over previous
import jax
import jax.numpy as jnp
from jax import lax
from jax.experimental import pallas as pl
from jax.experimental.pallas import tpu as pltpu

N_DEV = 4
MESH = pl.DeviceIdType.MESH


def kernel(x, Wq, Wo, K_ext, V_ext):
    B, Sq, D = x.shape
    _, skv, Hq, Dh = K_ext.shape
    Hh = Hq // 2

    kT = jnp.transpose(K_ext, (2, 0, 3, 1))
    vT = jnp.transpose(V_ext, (2, 0, 3, 1))
    wqT = jnp.transpose(Wq.reshape(D, Hq, Dh), (1, 0, 2))
    woH = Wo.reshape(Hq, Dh, D)

    def body(x_ref, wq_ref, wo_ref, k_ref, v_ref, out_ref,
             kfull, vfull, send_sems, recv_sems):
        my = lax.axis_index("i")
        left = lax.rem(my + (N_DEV - 1), N_DEV)
        right = lax.rem(my + 1, N_DEV)

        barrier = pltpu.get_barrier_semaphore()
        for nbr in (left, right):
            pl.semaphore_signal(barrier, inc=1, device_id=(nbr,),
                                device_id_type=MESH)
        pl.semaphore_wait(barrier, 2)

        kfull[0] = k_ref[...].astype(jnp.bfloat16)
        vfull[0] = v_ref[...].astype(jnp.bfloat16)

        def rdma(i, src, dst, dev):
            return pltpu.make_async_remote_copy(
                src_ref=src, dst_ref=dst,
                send_sem=send_sems.at[i], recv_sem=recv_sems.at[i],
                device_id=(dev,), device_id_type=MESH)

        r0 = rdma(0, kfull.at[0], kfull.at[1], right)
        r1 = rdma(1, vfull.at[0], vfull.at[1], right)
        r2 = rdma(2, kfull.at[0], kfull.at[2], left)
        r3 = rdma(3, vfull.at[0], vfull.at[2], left)
        for r in (r0, r1, r2, r3):
            r.start()

        xb = [x_ref[b].astype(jnp.bfloat16) for b in range(B)]
        qs = [[jnp.dot(xb[b], wq_ref[hh].astype(jnp.bfloat16),
                       preferred_element_type=jnp.float32
                       ).astype(jnp.bfloat16)
               for hh in range(Hq)] for b in range(B)]

        def chunk_update(slot, state):
            new = []
            i = 0
            for b in range(B):
                for hh in range(Hq):
                    kT_bh = kfull[slot, hh, b]
                    vT_bh = vfull[slot, hh, b]
                    s = lax.dot_general(
                        qs[b][hh], kT_bh, (((1,), (0,)), ((), ())),
                        preferred_element_type=jnp.float32) * 0.125
                    mj = jnp.max(s, axis=-1, keepdims=True)
                    if state is None:
                        p = jnp.exp(s - mj)
                        l = jnp.sum(p, axis=-1, keepdims=True)
                        o = lax.dot_general(
                            p.astype(jnp.bfloat16), vT_bh,
                            (((1,), (1,)), ((), ())),
                            preferred_element_type=jnp.float32)
                        new.append((mj, l, o))
                    else:
                        m0, l0, o0 = state[i]
                        mn = jnp.maximum(m0, mj)
                        alpha = jnp.exp(m0 - mn)
                        p = jnp.exp(s - mn)
                        l = l0 * alpha + jnp.sum(p, axis=-1, keepdims=True)
                        o = o0 * alpha + lax.dot_general(
                            p.astype(jnp.bfloat16), vT_bh,
                            (((1,), (1,)), ((), ())),
                            preferred_element_type=jnp.float32)
                        new.append((mn, l, o))
                    i += 1
            return new

        state = chunk_update(0, None)

        r0.wait_recv()
        r1.wait_recv()
        r4 = rdma(4, kfull.at[1, 0:Hh], kfull.at[3, 0:Hh], right)
        r5 = rdma(5, vfull.at[1, 0:Hh], vfull.at[3, 0:Hh], right)
        r4.start()
        r5.start()
        r2.wait_recv()
        r3.wait_recv()
        r6 = rdma(6, kfull.at[2, Hh:Hq], kfull.at[3, Hh:Hq], left)
        r7 = rdma(7, vfull.at[2, Hh:Hq], vfull.at[3, Hh:Hq], left)
        r6.start()
        r7.start()

        state = chunk_update(1, state)
        state = chunk_update(2, state)

        for r in (r4, r5, r6, r7):
            r.wait_recv()
        state = chunk_update(3, state)

        i = 0
        for b in range(B):
            acc = jnp.zeros((Sq, D), jnp.float32)
            for hh in range(Hq):
                m, l, o = state[i]
                acc = acc + lax.dot_general(
                    (o / l).astype(jnp.bfloat16),
                    wo_ref[hh].astype(jnp.bfloat16),
                    (((1,), (0,)), ((), ())),
                    preferred_element_type=jnp.float32)
                i += 1
            out_ref[b] = acc

        for r in (r0, r1, r2, r3, r4, r5, r6, r7):
            r.wait_send()

    return pl.pallas_call(
        body,
        out_shape=jax.ShapeDtypeStruct((B, Sq, D), jnp.float32),
        in_specs=[pl.BlockSpec(memory_space=pltpu.VMEM)] * 5,
        out_specs=pl.BlockSpec(memory_space=pltpu.VMEM),
        scratch_shapes=[
            pltpu.VMEM((N_DEV, Hq, B, Dh, skv), jnp.bfloat16),
            pltpu.VMEM((N_DEV, Hq, B, Dh, skv), jnp.bfloat16),
            pltpu.SemaphoreType.DMA((8,)),
            pltpu.SemaphoreType.DMA((8,)),
        ],
        compiler_params=pltpu.CompilerParams(collective_id=0),
    )(x, wqT, woH, kT, vT)
